# baseline (speedup 1.0000x reference)
import functools

import jax
import jax.numpy as jnp
from jax import lax
from jax.experimental import pallas as pl
from jax.experimental.pallas import tpu as pltpu
from jax.experimental.pallas import tpu_sc as plsc

N = 10000
D = 256
E = 160000
H = 128
NC = 2
NS = 16
EPT = E // NS
B = 128
NCHUNK = 80
EPT_PAD = NCHUNK * B
HS = NCHUNK // 2
NPAD = 10240
ROWS_PT = NPAD // NS
RB = 1000
GRID = N // RB


def _tc_lin_body(a_ref, hin_ref, wt_ref, b_ref, t_ref, lin0_ref, lin1_ref,
                 hout_ref, *, do_relu):
    h0 = a_ref[0]
    h1 = a_ref[1]
    if do_relu:
        h0 = jnp.maximum(h0, 0.0)
        h1 = jnp.maximum(h1, 0.0)
    lin0_ref[...] = (jnp.dot(h0, wt_ref[0, 0], preferred_element_type=jnp.float32)
                     + jnp.dot(h1, wt_ref[1, 0], preferred_element_type=jnp.float32)
                     + b_ref[0])
    lin1_ref[...] = (jnp.dot(h0, wt_ref[0, 1], preferred_element_type=jnp.float32)
                     + jnp.dot(h1, wt_ref[1, 1], preferred_element_type=jnp.float32)
                     + b_ref[1])
    t = t_ref[0, 0]
    hout_ref[0] = hin_ref[0] + t * h0
    hout_ref[1] = hin_ref[1] + t * h1


def _tc_lin(a, hin, wt4, b2, t, do_relu):
    return pl.pallas_call(
        functools.partial(_tc_lin_body, do_relu=do_relu),
        grid=(GRID,),
        in_specs=[
            pl.BlockSpec((2, RB, H), lambda i: (0, i, 0)),
            pl.BlockSpec((2, RB, H), lambda i: (0, i, 0)),
            pl.BlockSpec((2, 2, H, H), lambda i: (0, 0, 0, 0)),
            pl.BlockSpec((2, 1, H), lambda i: (0, 0, 0)),
            pl.BlockSpec((1, 1), lambda i: (0, 0)),
        ],
        out_specs=[
            pl.BlockSpec((RB, H), lambda i: (i, 0)),
            pl.BlockSpec((RB, H), lambda i: (i, 0)),
            pl.BlockSpec((2, RB, H), lambda i: (0, i, 0)),
        ],
        out_shape=[
            jax.ShapeDtypeStruct((N, H), jnp.float32),
            jax.ShapeDtypeStruct((N, H), jnp.float32),
            jax.ShapeDtypeStruct((2, N, H), jnp.float32),
        ],
    )(a, hin, wt4, b2, t)


def _tc_final_body(a_ref, hin_ref, t_ref, out_ref):
    t = t_ref[0, 0]
    out_ref[:, 0:H] = hin_ref[0] + t * jnp.maximum(a_ref[0], 0.0)
    out_ref[:, H:D] = hin_ref[1] + t * jnp.maximum(a_ref[1], 0.0)


def _tc_final(a, hin, t):
    return pl.pallas_call(
        _tc_final_body,
        grid=(GRID,),
        in_specs=[
            pl.BlockSpec((2, RB, H), lambda i: (0, i, 0)),
            pl.BlockSpec((2, RB, H), lambda i: (0, i, 0)),
            pl.BlockSpec((1, 1), lambda i: (0, 0)),
        ],
        out_specs=pl.BlockSpec((RB, D), lambda i: (i, 0)),
        out_shape=jax.ShapeDtypeStruct((N, D), jnp.float32),
    )(a, hin, t)


def _sc_agg_body(lin0_hbm, lin1_hbm, src_hbm, dst_hbm, w_hbm, zero_hbm,
                 out_hbm, acc, src_v, dst_v, w_v, buf0, buf1,
                 sem_g0, sem_g1, sem_s0, sem_s1):
    c = lax.axis_index("c")
    s = lax.axis_index("s")

    pltpu.sync_copy(zero_hbm.at[pl.ds(s * ROWS_PT, ROWS_PT)],
                    acc.at[pl.ds(s * ROWS_PT, ROWS_PT)])
    plsc.subcore_barrier()

    def issue_gather(g, buf, sem):
        @pl.when(c == 0)
        def _():
            pltpu.async_copy(lin0_hbm.at[src_v.at[g]], buf, sem)

        @pl.when(c == 1)
        def _():
            pltpu.async_copy(lin1_hbm.at[src_v.at[g]], buf, sem)

    def wait_gather(buf, sem):
        pltpu.make_async_copy(lin0_hbm.at[src_v.at[0]], buf, sem).wait()

    def issue_scatter(g, buf, sem):
        pltpu.async_copy(buf, acc.at[dst_v.at[g]], sem, add=True)

    def wait_scatter(g, buf, sem):
        pltpu.make_async_copy(buf, acc.at[dst_v.at[g]], sem).wait()

    def scale(buf, g):
        def scale16(i16, carry):
            wvec = w_v[g, pl.ds(i16 * 16, 16)]
            for k in range(16):
                wi = wvec[k]
                r = i16 * 16 + k
                for j in range(H // 16):
                    sl = pl.ds(j * 16, 16)
                    buf[r, sl] = buf[r, sl] * wi
            return carry

        lax.fori_loop(0, B // 16, scale16, 0)

    for stage in range(2):
        pltpu.sync_copy(src_hbm.at[s, pl.ds(stage * HS, HS)], src_v)
        pltpu.sync_copy(dst_hbm.at[s, pl.ds(stage * HS, HS)], dst_v)
        pltpu.sync_copy(w_hbm.at[s, pl.ds(stage * HS, HS)], w_v)

        issue_gather(0, buf0, sem_g0)
        issue_gather(1, buf1, sem_g1)

        def pair(k, carry):
            g0 = 2 * k
            g1 = 2 * k + 1
            wait_gather(buf0, sem_g0)
            scale(buf0, g0)
            issue_scatter(g0, buf0, sem_s0)
            wait_gather(buf1, sem_g1)
            scale(buf1, g1)
            issue_scatter(g1, buf1, sem_s1)
            wait_scatter(g0, buf0, sem_s0)

            @pl.when(k < HS // 2 - 1)
            def _():
                issue_gather(g0 + 2, buf0, sem_g0)

            wait_scatter(g1, buf1, sem_s1)

            @pl.when(k < HS // 2 - 1)
            def _():
                issue_gather(g1 + 2, buf1, sem_g1)

            return carry

        lax.fori_loop(0, HS // 2, pair, 0)

    plsc.subcore_barrier()

    pltpu.sync_copy(acc.at[pl.ds(s * ROWS_PT, ROWS_PT)],
                    out_hbm.at[c, pl.ds(s * ROWS_PT, ROWS_PT)])


@functools.cache
def _make_sc_agg():
    return pl.kernel(
        _sc_agg_body,
        out_type=jax.ShapeDtypeStruct((2, NPAD, H), jnp.float32),
        mesh=plsc.VectorSubcoreMesh(core_axis_name="c", subcore_axis_name="s",
                                    num_cores=NC, num_subcores=NS),
        scratch_types=[
            pltpu.VMEM_SHARED((NPAD, H), jnp.float32),
            pltpu.VMEM((HS, B), jnp.int32),
            pltpu.VMEM((HS, B), jnp.int32),
            pltpu.VMEM((HS, B), jnp.float32),
            pltpu.VMEM((B, H), jnp.float32),
            pltpu.VMEM((B, H), jnp.float32),
            pltpu.SemaphoreType.DMA,
            pltpu.SemaphoreType.DMA,
            pltpu.SemaphoreType.DMA,
            pltpu.SemaphoreType.DMA,
        ],
    )


def kernel(x, edge_index, edge_weight, W0, b0, W1, b1, W2, b2, temp):
    pad = ((0, 0), (0, EPT_PAD - EPT))
    src = jnp.pad(edge_index[0].astype(jnp.int32).reshape(NS, EPT),
                  pad).reshape(NS, NCHUNK, B)
    dst = jnp.pad(edge_index[1].astype(jnp.int32).reshape(NS, EPT),
                  pad, constant_values=N).reshape(NS, NCHUNK, B)
    w = jnp.pad(edge_weight.reshape(NS, EPT), pad).reshape(NS, NCHUNK, B)
    zeros_h = jnp.zeros((NPAD, H), jnp.float32)

    x2 = x.reshape(N, 2, H).transpose(1, 0, 2)
    hid = jnp.zeros((2, N, H), jnp.float32)

    a = x2
    for i, (W, b) in enumerate(((W0, b0), (W1, b1), (W2, b2))):
        wt4 = W.T.reshape(2, H, 2, H).transpose(0, 2, 1, 3)
        b2_ = b.reshape(2, 1, H)
        t = temp[i].reshape(1, 1)
        lin0, lin1, hid = _tc_lin(a, hid, wt4, b2_, t, do_relu=(i > 0))
        a = _make_sc_agg()(lin0, lin1, src, dst, w, zeros_h)

    return _tc_final(a, hid, temp[3].reshape(1, 1))


# speedup vs baseline: 1.0197x; 1.0197x over previous
import functools

import jax
import jax.numpy as jnp
from jax import lax
from jax.experimental import pallas as pl
from jax.experimental.pallas import tpu as pltpu
from jax.experimental.pallas import tpu_sc as plsc

N = 10000
D = 256
E = 160000
H = 128
NC = 2
NS = 16
EPT = E // NS
B = 128
NCHUNK = 79
EPT_PAD = NCHUNK * B
NPAD = 10240
ROWS_PT = NPAD // NS
RB = 1000
GRID = N // RB


def _tc_lin_body(a_ref, hin_ref, wt_ref, b_ref, t_ref, lin0_ref, lin1_ref,
                 hout_ref, *, do_relu):
    h0 = a_ref[0]
    h1 = a_ref[1]
    if do_relu:
        h0 = jnp.maximum(h0, 0.0)
        h1 = jnp.maximum(h1, 0.0)
    lin0_ref[...] = (jnp.dot(h0, wt_ref[0, 0], preferred_element_type=jnp.float32)
                     + jnp.dot(h1, wt_ref[1, 0], preferred_element_type=jnp.float32)
                     + b_ref[0])
    lin1_ref[...] = (jnp.dot(h0, wt_ref[0, 1], preferred_element_type=jnp.float32)
                     + jnp.dot(h1, wt_ref[1, 1], preferred_element_type=jnp.float32)
                     + b_ref[1])
    t = t_ref[0, 0]
    hout_ref[0] = hin_ref[0] + t * h0
    hout_ref[1] = hin_ref[1] + t * h1


def _tc_lin(a, hin, wt4, b2, t, do_relu):
    return pl.pallas_call(
        functools.partial(_tc_lin_body, do_relu=do_relu),
        grid=(GRID,),
        in_specs=[
            pl.BlockSpec((2, RB, H), lambda i: (0, i, 0)),
            pl.BlockSpec((2, RB, H), lambda i: (0, i, 0)),
            pl.BlockSpec((2, 2, H, H), lambda i: (0, 0, 0, 0)),
            pl.BlockSpec((2, 1, H), lambda i: (0, 0, 0)),
            pl.BlockSpec((1, 1), lambda i: (0, 0)),
        ],
        out_specs=[
            pl.BlockSpec((RB, H), lambda i: (i, 0)),
            pl.BlockSpec((RB, H), lambda i: (i, 0)),
            pl.BlockSpec((2, RB, H), lambda i: (0, i, 0)),
        ],
        out_shape=[
            jax.ShapeDtypeStruct((N, H), jnp.float32),
            jax.ShapeDtypeStruct((N, H), jnp.float32),
            jax.ShapeDtypeStruct((2, N, H), jnp.float32),
        ],
    )(a, hin, wt4, b2, t)


def _tc_final_body(a_ref, hin_ref, t_ref, out_ref):
    t = t_ref[0, 0]
    out_ref[:, 0:H] = hin_ref[0] + t * jnp.maximum(a_ref[0], 0.0)
    out_ref[:, H:D] = hin_ref[1] + t * jnp.maximum(a_ref[1], 0.0)


def _tc_final(a, hin, t):
    return pl.pallas_call(
        _tc_final_body,
        grid=(GRID,),
        in_specs=[
            pl.BlockSpec((2, RB, H), lambda i: (0, i, 0)),
            pl.BlockSpec((2, RB, H), lambda i: (0, i, 0)),
            pl.BlockSpec((1, 1), lambda i: (0, 0)),
        ],
        out_specs=pl.BlockSpec((RB, D), lambda i: (i, 0)),
        out_shape=jax.ShapeDtypeStruct((N, D), jnp.float32),
    )(a, hin, t)


def _sc_agg_body(lin0_hbm, lin1_hbm, src_hbm, dst_hbm, w_hbm, zero_hbm,
                 out_hbm, acc, src_v, dst_v, w_v, rows_v, sem):
    c = lax.axis_index("c")
    s = lax.axis_index("s")

    pltpu.sync_copy(zero_hbm.at[pl.ds(s * ROWS_PT, ROWS_PT)],
                    acc.at[pl.ds(s * ROWS_PT, ROWS_PT)])

    pltpu.sync_copy(src_hbm.at[s], src_v)
    pltpu.sync_copy(dst_hbm.at[s], dst_v)
    pltpu.sync_copy(w_hbm.at[s], w_v)
    plsc.subcore_barrier()

    def chunk(g, carry):
        @pl.when(c == 0)
        def _():
            pltpu.async_copy(lin0_hbm.at[src_v.at[g]], rows_v, sem).wait()

        @pl.when(c == 1)
        def _():
            pltpu.async_copy(lin1_hbm.at[src_v.at[g]], rows_v, sem).wait()

        def scale16(i16, carry2):
            wvec = w_v[g, pl.ds(i16 * 16, 16)]
            for k in range(16):
                wi = wvec[k]
                r = i16 * 16 + k
                for j in range(H // 16):
                    sl = pl.ds(j * 16, 16)
                    rows_v[r, sl] = rows_v[r, sl] * wi
            return carry2

        lax.fori_loop(0, B // 16, scale16, 0)

        pltpu.sync_copy(rows_v, acc.at[dst_v.at[g]], add=True)
        return carry

    lax.fori_loop(0, NCHUNK, chunk, 0)
    plsc.subcore_barrier()

    pltpu.sync_copy(acc.at[pl.ds(s * ROWS_PT, ROWS_PT)],
                    out_hbm.at[c, pl.ds(s * ROWS_PT, ROWS_PT)])


@functools.cache
def _make_sc_agg():
    return pl.kernel(
        _sc_agg_body,
        out_type=jax.ShapeDtypeStruct((2, NPAD, H), jnp.float32),
        mesh=plsc.VectorSubcoreMesh(core_axis_name="c", subcore_axis_name="s",
                                    num_cores=NC, num_subcores=NS),
        scratch_types=[
            pltpu.VMEM_SHARED((NPAD, H), jnp.float32),
            pltpu.VMEM((NCHUNK, B), jnp.int32),
            pltpu.VMEM((NCHUNK, B), jnp.int32),
            pltpu.VMEM((NCHUNK, B), jnp.float32),
            pltpu.VMEM((B, H), jnp.float32),
            pltpu.SemaphoreType.DMA,
        ],
    )


def kernel(x, edge_index, edge_weight, W0, b0, W1, b1, W2, b2, temp):
    pad = ((0, 0), (0, EPT_PAD - EPT))
    src = jnp.pad(edge_index[0].astype(jnp.int32).reshape(NS, EPT),
                  pad).reshape(NS, NCHUNK, B)
    dst = jnp.pad(edge_index[1].astype(jnp.int32).reshape(NS, EPT),
                  pad, constant_values=N).reshape(NS, NCHUNK, B)
    w = jnp.pad(edge_weight.reshape(NS, EPT), pad).reshape(NS, NCHUNK, B)
    zeros_h = jnp.zeros((NPAD, H), jnp.float32)

    x2 = x.reshape(N, 2, H).transpose(1, 0, 2)
    hid = jnp.zeros((2, N, H), jnp.float32)

    a = x2
    for i, (W, b) in enumerate(((W0, b0), (W1, b1), (W2, b2))):
        wt4 = W.T.reshape(2, H, 2, H).transpose(0, 2, 1, 3)
        b2_ = b.reshape(2, 1, H)
        t = temp[i].reshape(1, 1)
        lin0, lin1, hid = _tc_lin(a, hid, wt4, b2_, t, do_relu=(i > 0))
        a = _make_sc_agg()(lin0, lin1, src, dst, w, zeros_h)

    return _tc_final(a, hid, temp[3].reshape(1, 1))
